# baseline (speedup 1.0000x reference)
import jax
import jax.numpy as jnp
from jax import lax
from jax.experimental import pallas as pl
from jax.experimental.pallas import tpu as pltpu
from jax.experimental.pallas import tpu_sc as plsc

B = 16384
D = 64
NC = 2
NS = 16
NW = NC * NS
BPW = B // NW
CHUNK = 128
NCH = BPW // CHUNK
GPC = CHUNK // 16

N_U = 1000000
N_I = 100000
CB_U = (N_U + 127) // 128
CB_I = (N_I + 127) // 128


def _repack_body(ut_t, it_t, u2_out, i2_out, colbuf, rowbuf):
    c = lax.axis_index("c")
    s = lax.axis_index("s")
    wid = s * NC + c

    iota16 = lax.iota(jnp.int32, 16)

    def make_col_loop(src, dst, n_cols, n_rows):
        def col_body(t, _):
            cb = wid + t * NW

            @pl.when(cb < n_cols)
            def _():
                start = pl.multiple_of(cb * 128, 128)
                pltpu.sync_copy(src.at[:, pl.ds(start, 128)], colbuf)
                def l_body(l, _):
                    lanes = jnp.broadcast_to(l, (16,)).astype(jnp.int32)
                    v = l >> 1
                    hbase = (l & 1) * D
                    for k in range(D // 16):
                        vals = plsc.load_gather(colbuf,
                                                [k * 16 + iota16, lanes])
                        rowbuf[v, pl.ds(hbase + k * 16, 16)] = vals
                    return 0

                lax.fori_loop(0, 128, l_body, 0)
                vfirst = cb * 64
                nv = jnp.minimum(n_rows // 2 - vfirst, 64)
                for q in range(4):
                    @pl.when(nv >= (q + 1) * 16)
                    def _(q=q):
                        pltpu.sync_copy(rowbuf.at[pl.ds(q * 16, 16)],
                                        dst.at[pl.ds(vfirst + q * 16, 16)])
            return 0

        return col_body

    lax.fori_loop(0, (CB_U + NW - 1) // NW,
                  make_col_loop(ut_t, u2_out, CB_U, N_U), 0)
    lax.fori_loop(0, (CB_I + NW - 1) // NW,
                  make_col_loop(it_t, i2_out, CB_I, N_I), 0)


@jax.jit
def _repack(ut_t, it_t):
    mesh = plsc.VectorSubcoreMesh(core_axis_name="c", subcore_axis_name="s",
                                  num_cores=NC, num_subcores=NS)
    f32 = jnp.float32
    run = pl.kernel(
        _repack_body,
        out_type=(jax.ShapeDtypeStruct((N_U // 2, 128), f32),
                  jax.ShapeDtypeStruct((N_I // 2, 128), f32)),
        mesh=mesh,
        scratch_types=[
            pltpu.VMEM((D, 128), f32),
            pltpu.VMEM((D, 128), f32),
        ],
        compiler_params=pltpu.CompilerParams(
            needs_layout_passes=False,
            use_tc_tiling_on_sc=True,
            disable_bounds_checks=True),
    )
    return run(ut_t, it_t)


def _bprmf_body(users_r, pos_r, neg_r, ut_r, it_r,
                pos_s_out, neg_s_out, u_out, p_out, n_out,
                idx_u, idx_p, idx_n, vr_u, vr_p, vr_n,
                u_rows, p_rows, n_rows, sc_p, sc_n, sem):
    c = lax.axis_index("c")
    s = lax.axis_index("s")
    wid = s * NC + c
    base = wid * BPW

    pltpu.sync_copy(users_r.at[pl.ds(wid * NCH, NCH)], idx_u)
    pltpu.sync_copy(pos_r.at[pl.ds(wid * NCH, NCH)], idx_p)
    pltpu.sync_copy(neg_r.at[pl.ds(wid * NCH, NCH)], idx_n)

    for ch in range(NCH):
        for g in range(GPC):
            sl = pl.ds(g * 16, 16)
            vr_u[ch, sl] = idx_u[ch, sl] >> 1
            vr_p[ch, sl] = idx_p[ch, sl] >> 1
            vr_n[ch, sl] = idx_n[ch, sl] >> 1

    iota16 = lax.iota(jnp.int32, 16)
    zero16 = jnp.zeros((16,), jnp.float32)

    for ch in range(NCH):
        cps = (pltpu.async_copy(ut_r.at[vr_u.at[ch]], u_rows, sem),
               pltpu.async_copy(it_r.at[vr_p.at[ch]], p_rows, sem),
               pltpu.async_copy(it_r.at[vr_n.at[ch]], n_rows, sem))
        for cp in cps:
            cp.wait()

        for g in range(GPC):
            sl = pl.ds(g * 16, 16)
            rows_e = g * 16 + iota16
            hu = (idx_u[ch, sl] & 1) * D
            hp = (idx_p[ch, sl] & 1) * D
            hn = (idx_n[ch, sl] & 1) * D

            def dbody(d, carry, rows_e=rows_e, hu=hu, hp=hp, hn=hn):
                ap, an = carry
                uc = plsc.load_gather(u_rows, [rows_e, hu + d])
                pc = plsc.load_gather(p_rows, [rows_e, hp + d])
                nc = plsc.load_gather(n_rows, [rows_e, hn + d])
                return (ap + uc * pc, an + uc * nc)

            ap, an = lax.fori_loop(0, D, dbody, (zero16, zero16))
            osl = pl.ds(ch * CHUNK + g * 16, 16)
            sc_p[osl] = ap
            sc_n[osl] = an

        out_sl = pl.ds(base + ch * CHUNK, CHUNK)
        pltpu.sync_copy(u_rows, u_out.at[out_sl])
        pltpu.sync_copy(p_rows, p_out.at[out_sl])
        pltpu.sync_copy(n_rows, n_out.at[out_sl])

    out_sl = pl.ds(base, BPW)
    pltpu.sync_copy(sc_p, pos_s_out.at[out_sl])
    pltpu.sync_copy(sc_n, neg_s_out.at[out_sl])


@jax.jit
def _bprmf(users2, pos2, neg2, ut2, it2):
    mesh = plsc.VectorSubcoreMesh(core_axis_name="c", subcore_axis_name="s",
                                  num_cores=NC, num_subcores=NS)
    f32 = jnp.float32
    out_type = (
        jax.ShapeDtypeStruct((B,), f32),
        jax.ShapeDtypeStruct((B,), f32),
        jax.ShapeDtypeStruct((B, 2 * D), f32),
        jax.ShapeDtypeStruct((B, 2 * D), f32),
        jax.ShapeDtypeStruct((B, 2 * D), f32),
    )
    i32 = jnp.int32
    scratch = [
        pltpu.VMEM((NCH, CHUNK), i32),
        pltpu.VMEM((NCH, CHUNK), i32),
        pltpu.VMEM((NCH, CHUNK), i32),
        pltpu.VMEM((NCH, CHUNK), i32),
        pltpu.VMEM((NCH, CHUNK), i32),
        pltpu.VMEM((NCH, CHUNK), i32),
        pltpu.VMEM((CHUNK, 2 * D), f32),
        pltpu.VMEM((CHUNK, 2 * D), f32),
        pltpu.VMEM((CHUNK, 2 * D), f32),
        pltpu.VMEM((BPW,), f32),
        pltpu.VMEM((BPW,), f32),
        pltpu.SemaphoreType.DMA,
    ]
    run = pl.kernel(_bprmf_body, out_type=out_type, mesh=mesh,
                    scratch_types=scratch,
                    compiler_params=pltpu.CompilerParams(
                        needs_layout_passes=False,
                        use_tc_tiling_on_sc=True))
    return run(users2, pos2, neg2, ut2, it2)


def kernel(users, pos_items, neg_items, user_table, item_table):
    users2 = users.astype(jnp.int32).reshape(NW * NCH, CHUNK)
    pos2 = pos_items.astype(jnp.int32).reshape(NW * NCH, CHUNK)
    neg2 = neg_items.astype(jnp.int32).reshape(NW * NCH, CHUNK)
    ut2, it2 = _repack(user_table.T, item_table.T)
    ps, ns, uv, pv, nv = _bprmf(users2, pos2, neg2, ut2, it2)
    u_odd = (users.astype(jnp.int32) & 1)[:, None] == 1
    p_odd = (pos_items.astype(jnp.int32) & 1)[:, None] == 1
    n_odd = (neg_items.astype(jnp.int32) & 1)[:, None] == 1
    u_emb = jnp.where(u_odd, uv[:, D:], uv[:, :D])
    pos_emb = jnp.where(p_odd, pv[:, D:], pv[:, :D])
    neg_emb = jnp.where(n_odd, nv[:, D:], nv[:, :D])
    return (ps, ns, u_emb, pos_emb, neg_emb)


# speedup vs baseline: 1.1002x; 1.1002x over previous
import jax
import jax.numpy as jnp
from jax import lax
from jax.experimental import pallas as pl
from jax.experimental.pallas import tpu as pltpu
from jax.experimental.pallas import tpu_sc as plsc

B = 16384
D = 64
NC = 2
NS = 16
NW = NC * NS
BPW = B // NW
CHUNK = 128
NCH = BPW // CHUNK
GPC = CHUNK // 16

N_U = 1000000
N_I = 100000
CB_U = (N_U + 127) // 128
CB_I = (N_I + 127) // 128


def _transpose_stripe(colbuf, rowbuf):
    iota16 = lax.iota(jnp.int32, 16)
    rows_k = [k * 16 + iota16 for k in range(D // 16)]

    def grp_body(g, _):
        lbase = g * 16
        for j in range(16):
            l = lbase + j
            lanes = jnp.broadcast_to(l, (16,)).astype(jnp.int32)
            v = l >> 1
            hbase = (j & 1) * D
            for k in range(D // 16):
                vals = plsc.load_gather(colbuf, [rows_k[k], lanes])
                rowbuf[v, pl.ds(hbase + k * 16, 16)] = vals
        return 0

    lax.fori_loop(0, 8, grp_body, 0)


def _repack_body(ut_t, it_t, u2_out, i2_out,
                 colA, colB, rowA, rowB, semA, semB):
    c = lax.axis_index("c")
    s = lax.axis_index("s")
    wid = s * NC + c

    def make_pair_loop(src, dst, n_full):
        def pair_body(tp, _):
            cb0 = wid + (2 * tp) * NW
            cb1 = wid + (2 * tp + 1) * NW
            cb0c = jnp.minimum(cb0, n_full - 1)
            cb1c = jnp.minimum(cb1, n_full - 1)
            st0 = pl.multiple_of(cb0c * 128, 128)
            st1 = pl.multiple_of(cb1c * 128, 128)
            ci0 = pltpu.async_copy(src.at[:, pl.ds(st0, 128)], colA, semA)
            ci1 = pltpu.async_copy(src.at[:, pl.ds(st1, 128)], colB, semB)
            ci0.wait()
            _transpose_stripe(colA, rowA)
            co0 = pltpu.async_copy(rowA, dst.at[pl.ds(cb0c * 64, 64)], semA)
            ci1.wait()
            _transpose_stripe(colB, rowB)
            co1 = pltpu.async_copy(rowB, dst.at[pl.ds(cb1c * 64, 64)], semB)
            co0.wait()
            co1.wait()
            return 0

        return pair_body

    n_full_u = N_U // 128
    n_full_i = N_I // 128
    lax.fori_loop(0, (n_full_u + 2 * NW - 1) // (2 * NW),
                  make_pair_loop(ut_t, u2_out, n_full_u), 0)
    lax.fori_loop(0, (n_full_i + 2 * NW - 1) // (2 * NW),
                  make_pair_loop(it_t, i2_out, n_full_i), 0)

    def tail(src, dst, cb, n_rows, who):
        @pl.when(wid == who)
        def _():
            start = pl.multiple_of(cb * 128, 128)
            pltpu.sync_copy(src.at[:, pl.ds(start, 128)], colA)
            _transpose_stripe(colA, rowA)
            nv = n_rows // 2 - cb * 64
            for q in range(4):
                if True:
                    @pl.when(nv >= (q + 1) * 16)
                    def _(q=q):
                        pltpu.sync_copy(rowA.at[pl.ds(q * 16, 16)],
                                        dst.at[pl.ds(cb * 64 + q * 16, 16)])

    tail(ut_t, u2_out, n_full_u, N_U, 0)
    tail(it_t, i2_out, n_full_i, N_I, 1)


@jax.jit
def _repack(ut_t, it_t):
    mesh = plsc.VectorSubcoreMesh(core_axis_name="c", subcore_axis_name="s",
                                  num_cores=NC, num_subcores=NS)
    f32 = jnp.float32
    run = pl.kernel(
        _repack_body,
        out_type=(jax.ShapeDtypeStruct((N_U // 2, 128), f32),
                  jax.ShapeDtypeStruct((N_I // 2, 128), f32)),
        mesh=mesh,
        scratch_types=[
            pltpu.VMEM((D, 128), f32),
            pltpu.VMEM((D, 128), f32),
            pltpu.VMEM((D, 128), f32),
            pltpu.VMEM((D, 128), f32),
            pltpu.SemaphoreType.DMA,
            pltpu.SemaphoreType.DMA,
        ],
        compiler_params=pltpu.CompilerParams(
            needs_layout_passes=False,
            use_tc_tiling_on_sc=True,
            disable_bounds_checks=True),
    )
    return run(ut_t, it_t)


def _bprmf_body(users_r, pos_r, neg_r, ut_r, it_r,
                pos_s_out, neg_s_out, u_out, p_out, n_out,
                idx_u, idx_p, idx_n, vr_u, vr_p, vr_n,
                u_rows, p_rows, n_rows, sc_p, sc_n, sem):
    c = lax.axis_index("c")
    s = lax.axis_index("s")
    wid = s * NC + c
    base = wid * BPW

    pltpu.sync_copy(users_r.at[pl.ds(wid * NCH, NCH)], idx_u)
    pltpu.sync_copy(pos_r.at[pl.ds(wid * NCH, NCH)], idx_p)
    pltpu.sync_copy(neg_r.at[pl.ds(wid * NCH, NCH)], idx_n)

    for ch in range(NCH):
        for g in range(GPC):
            sl = pl.ds(g * 16, 16)
            vr_u[ch, sl] = idx_u[ch, sl] >> 1
            vr_p[ch, sl] = idx_p[ch, sl] >> 1
            vr_n[ch, sl] = idx_n[ch, sl] >> 1

    iota16 = lax.iota(jnp.int32, 16)
    zero16 = jnp.zeros((16,), jnp.float32)

    for ch in range(NCH):
        cps = (pltpu.async_copy(ut_r.at[vr_u.at[ch]], u_rows, sem),
               pltpu.async_copy(it_r.at[vr_p.at[ch]], p_rows, sem),
               pltpu.async_copy(it_r.at[vr_n.at[ch]], n_rows, sem))
        for cp in cps:
            cp.wait()

        for g in range(GPC):
            sl = pl.ds(g * 16, 16)
            rows_e = g * 16 + iota16
            hu = (idx_u[ch, sl] & 1) * D
            hp = (idx_p[ch, sl] & 1) * D
            hn = (idx_n[ch, sl] & 1) * D

            def dbody(d, carry, rows_e=rows_e, hu=hu, hp=hp, hn=hn):
                ap, an = carry
                uc = plsc.load_gather(u_rows, [rows_e, hu + d])
                pc = plsc.load_gather(p_rows, [rows_e, hp + d])
                nc = plsc.load_gather(n_rows, [rows_e, hn + d])
                return (ap + uc * pc, an + uc * nc)

            ap, an = lax.fori_loop(0, D, dbody, (zero16, zero16))
            osl = pl.ds(ch * CHUNK + g * 16, 16)
            sc_p[osl] = ap
            sc_n[osl] = an

        out_sl = pl.ds(base + ch * CHUNK, CHUNK)
        pltpu.sync_copy(u_rows, u_out.at[out_sl])
        pltpu.sync_copy(p_rows, p_out.at[out_sl])
        pltpu.sync_copy(n_rows, n_out.at[out_sl])

    out_sl = pl.ds(base, BPW)
    pltpu.sync_copy(sc_p, pos_s_out.at[out_sl])
    pltpu.sync_copy(sc_n, neg_s_out.at[out_sl])


@jax.jit
def _bprmf(users2, pos2, neg2, ut2, it2):
    mesh = plsc.VectorSubcoreMesh(core_axis_name="c", subcore_axis_name="s",
                                  num_cores=NC, num_subcores=NS)
    f32 = jnp.float32
    out_type = (
        jax.ShapeDtypeStruct((B,), f32),
        jax.ShapeDtypeStruct((B,), f32),
        jax.ShapeDtypeStruct((B, 2 * D), f32),
        jax.ShapeDtypeStruct((B, 2 * D), f32),
        jax.ShapeDtypeStruct((B, 2 * D), f32),
    )
    i32 = jnp.int32
    scratch = [
        pltpu.VMEM((NCH, CHUNK), i32),
        pltpu.VMEM((NCH, CHUNK), i32),
        pltpu.VMEM((NCH, CHUNK), i32),
        pltpu.VMEM((NCH, CHUNK), i32),
        pltpu.VMEM((NCH, CHUNK), i32),
        pltpu.VMEM((NCH, CHUNK), i32),
        pltpu.VMEM((CHUNK, 2 * D), f32),
        pltpu.VMEM((CHUNK, 2 * D), f32),
        pltpu.VMEM((CHUNK, 2 * D), f32),
        pltpu.VMEM((BPW,), f32),
        pltpu.VMEM((BPW,), f32),
        pltpu.SemaphoreType.DMA,
    ]
    run = pl.kernel(_bprmf_body, out_type=out_type, mesh=mesh,
                    scratch_types=scratch,
                    compiler_params=pltpu.CompilerParams(
                        needs_layout_passes=False,
                        use_tc_tiling_on_sc=True))
    return run(users2, pos2, neg2, ut2, it2)


def kernel(users, pos_items, neg_items, user_table, item_table):
    users2 = users.astype(jnp.int32).reshape(NW * NCH, CHUNK)
    pos2 = pos_items.astype(jnp.int32).reshape(NW * NCH, CHUNK)
    neg2 = neg_items.astype(jnp.int32).reshape(NW * NCH, CHUNK)
    ut2, it2 = _repack(user_table.T, item_table.T)
    ps, ns, uv, pv, nv = _bprmf(users2, pos2, neg2, ut2, it2)
    u_odd = (users.astype(jnp.int32) & 1)[:, None] == 1
    p_odd = (pos_items.astype(jnp.int32) & 1)[:, None] == 1
    n_odd = (neg_items.astype(jnp.int32) & 1)[:, None] == 1
    u_emb = jnp.where(u_odd, uv[:, D:], uv[:, :D])
    pos_emb = jnp.where(p_odd, pv[:, D:], pv[:, :D])
    neg_emb = jnp.where(n_odd, nv[:, D:], nv[:, :D])
    return (ps, ns, u_emb, pos_emb, neg_emb)


# speedup vs baseline: 2.4699x; 2.2449x over previous
import jax
import jax.numpy as jnp
from jax import lax
from jax.experimental import pallas as pl
from jax.experimental.pallas import tpu as pltpu
from jax.experimental.pallas import tpu_sc as plsc

B = 16384
D = 64
NC = 2
NS = 16
NW = NC * NS
BPW = B // NW
CHUNK = 128
NCH = BPW // CHUNK
GPC = CHUNK // 16

N_U = 1000000
N_I = 100000
CB_U = (N_U + 127) // 128
CB_I = (N_I + 127) // 128


def _transpose_stripe(colbuf, rowbuf):
    iota16 = lax.iota(jnp.int32, 16)
    lanes_l0 = [l0 + iota16 for l0 in range(0, 128, 16)]
    v_l0 = [(l0 + iota16) >> 1 for l0 in range(0, 128, 16)]
    hb = (iota16 & 1) * D

    def r_body(r, _):
        rowoff = (iota16 + r) & 15
        for d0 in range(0, D, 16):
            rows = rowoff + d0
            cc = hb + rows
            for q in range(8):
                vals = plsc.load_gather(colbuf, [rows, lanes_l0[q]])
                plsc.store_scatter(rowbuf, [v_l0[q], cc], vals)
        return 0

    lax.fori_loop(0, 16, r_body, 0)


def _repack_body(ut_t, it_t, u2_out, i2_out,
                 colA, colB, rowA, rowB, semA, semB):
    c = lax.axis_index("c")
    s = lax.axis_index("s")
    wid = s * NC + c

    def make_pair_loop(src, dst, n_full):
        def pair_body(tp, _):
            cb0 = wid + (2 * tp) * NW
            cb1 = wid + (2 * tp + 1) * NW
            cb0c = jnp.minimum(cb0, n_full - 1)
            cb1c = jnp.minimum(cb1, n_full - 1)
            st0 = pl.multiple_of(cb0c * 128, 128)
            st1 = pl.multiple_of(cb1c * 128, 128)
            ci0 = pltpu.async_copy(src.at[:, pl.ds(st0, 128)], colA, semA)
            ci1 = pltpu.async_copy(src.at[:, pl.ds(st1, 128)], colB, semB)
            ci0.wait()
            _transpose_stripe(colA, rowA)
            co0 = pltpu.async_copy(rowA, dst.at[pl.ds(cb0c * 64, 64)], semA)
            ci1.wait()
            _transpose_stripe(colB, rowB)
            co1 = pltpu.async_copy(rowB, dst.at[pl.ds(cb1c * 64, 64)], semB)
            co0.wait()
            co1.wait()
            return 0

        return pair_body

    n_full_u = N_U // 128
    n_full_i = N_I // 128
    lax.fori_loop(0, (n_full_u + 2 * NW - 1) // (2 * NW),
                  make_pair_loop(ut_t, u2_out, n_full_u), 0)
    lax.fori_loop(0, (n_full_i + 2 * NW - 1) // (2 * NW),
                  make_pair_loop(it_t, i2_out, n_full_i), 0)

    def tail(src, dst, cb, n_rows, who):
        @pl.when(wid == who)
        def _():
            start = pl.multiple_of(cb * 128, 128)
            pltpu.sync_copy(src.at[:, pl.ds(start, 128)], colA)
            _transpose_stripe(colA, rowA)
            nv = n_rows // 2 - cb * 64
            for q in range(4):
                if True:
                    @pl.when(nv >= (q + 1) * 16)
                    def _(q=q):
                        pltpu.sync_copy(rowA.at[pl.ds(q * 16, 16)],
                                        dst.at[pl.ds(cb * 64 + q * 16, 16)])

    tail(ut_t, u2_out, n_full_u, N_U, 0)
    tail(it_t, i2_out, n_full_i, N_I, 1)


@jax.jit
def _repack(ut_t, it_t):
    mesh = plsc.VectorSubcoreMesh(core_axis_name="c", subcore_axis_name="s",
                                  num_cores=NC, num_subcores=NS)
    f32 = jnp.float32
    run = pl.kernel(
        _repack_body,
        out_type=(jax.ShapeDtypeStruct((N_U // 2, 128), f32),
                  jax.ShapeDtypeStruct((N_I // 2, 128), f32)),
        mesh=mesh,
        scratch_types=[
            pltpu.VMEM((D, 128), f32),
            pltpu.VMEM((D, 128), f32),
            pltpu.VMEM((D, 128), f32),
            pltpu.VMEM((D, 128), f32),
            pltpu.SemaphoreType.DMA,
            pltpu.SemaphoreType.DMA,
        ],
        compiler_params=pltpu.CompilerParams(
            needs_layout_passes=False,
            use_tc_tiling_on_sc=True,
            disable_bounds_checks=True),
    )
    return run(ut_t, it_t)


def _bprmf_body(users_r, pos_r, neg_r, ut_r, it_r,
                pos_s_out, neg_s_out, u_out, p_out, n_out,
                idx_u, idx_p, idx_n, vr_u, vr_p, vr_n,
                u_rows, p_rows, n_rows, sc_p, sc_n, sem):
    c = lax.axis_index("c")
    s = lax.axis_index("s")
    wid = s * NC + c
    base = wid * BPW

    pltpu.sync_copy(users_r.at[pl.ds(wid * NCH, NCH)], idx_u)
    pltpu.sync_copy(pos_r.at[pl.ds(wid * NCH, NCH)], idx_p)
    pltpu.sync_copy(neg_r.at[pl.ds(wid * NCH, NCH)], idx_n)

    for ch in range(NCH):
        for g in range(GPC):
            sl = pl.ds(g * 16, 16)
            vr_u[ch, sl] = idx_u[ch, sl] >> 1
            vr_p[ch, sl] = idx_p[ch, sl] >> 1
            vr_n[ch, sl] = idx_n[ch, sl] >> 1

    iota16 = lax.iota(jnp.int32, 16)
    zero16 = jnp.zeros((16,), jnp.float32)

    for ch in range(NCH):
        cps = (pltpu.async_copy(ut_r.at[vr_u.at[ch]], u_rows, sem),
               pltpu.async_copy(it_r.at[vr_p.at[ch]], p_rows, sem),
               pltpu.async_copy(it_r.at[vr_n.at[ch]], n_rows, sem))
        for cp in cps:
            cp.wait()

        for g in range(GPC):
            sl = pl.ds(g * 16, 16)
            rows_e = g * 16 + iota16
            hu = (idx_u[ch, sl] & 1) * D
            hp = (idx_p[ch, sl] & 1) * D
            hn = (idx_n[ch, sl] & 1) * D

            def dbody(d, carry, rows_e=rows_e, hu=hu, hp=hp, hn=hn):
                ap, an = carry
                uc = plsc.load_gather(u_rows, [rows_e, hu + d])
                pc = plsc.load_gather(p_rows, [rows_e, hp + d])
                nc = plsc.load_gather(n_rows, [rows_e, hn + d])
                return (ap + uc * pc, an + uc * nc)

            ap, an = lax.fori_loop(0, D, dbody, (zero16, zero16))
            osl = pl.ds(ch * CHUNK + g * 16, 16)
            sc_p[osl] = ap
            sc_n[osl] = an

        out_sl = pl.ds(base + ch * CHUNK, CHUNK)
        pltpu.sync_copy(u_rows, u_out.at[out_sl])
        pltpu.sync_copy(p_rows, p_out.at[out_sl])
        pltpu.sync_copy(n_rows, n_out.at[out_sl])

    out_sl = pl.ds(base, BPW)
    pltpu.sync_copy(sc_p, pos_s_out.at[out_sl])
    pltpu.sync_copy(sc_n, neg_s_out.at[out_sl])


@jax.jit
def _bprmf(users2, pos2, neg2, ut2, it2):
    mesh = plsc.VectorSubcoreMesh(core_axis_name="c", subcore_axis_name="s",
                                  num_cores=NC, num_subcores=NS)
    f32 = jnp.float32
    out_type = (
        jax.ShapeDtypeStruct((B,), f32),
        jax.ShapeDtypeStruct((B,), f32),
        jax.ShapeDtypeStruct((B, 2 * D), f32),
        jax.ShapeDtypeStruct((B, 2 * D), f32),
        jax.ShapeDtypeStruct((B, 2 * D), f32),
    )
    i32 = jnp.int32
    scratch = [
        pltpu.VMEM((NCH, CHUNK), i32),
        pltpu.VMEM((NCH, CHUNK), i32),
        pltpu.VMEM((NCH, CHUNK), i32),
        pltpu.VMEM((NCH, CHUNK), i32),
        pltpu.VMEM((NCH, CHUNK), i32),
        pltpu.VMEM((NCH, CHUNK), i32),
        pltpu.VMEM((CHUNK, 2 * D), f32),
        pltpu.VMEM((CHUNK, 2 * D), f32),
        pltpu.VMEM((CHUNK, 2 * D), f32),
        pltpu.VMEM((BPW,), f32),
        pltpu.VMEM((BPW,), f32),
        pltpu.SemaphoreType.DMA,
    ]
    run = pl.kernel(_bprmf_body, out_type=out_type, mesh=mesh,
                    scratch_types=scratch,
                    compiler_params=pltpu.CompilerParams(
                        needs_layout_passes=False,
                        use_tc_tiling_on_sc=True))
    return run(users2, pos2, neg2, ut2, it2)


def kernel(users, pos_items, neg_items, user_table, item_table):
    users2 = users.astype(jnp.int32).reshape(NW * NCH, CHUNK)
    pos2 = pos_items.astype(jnp.int32).reshape(NW * NCH, CHUNK)
    neg2 = neg_items.astype(jnp.int32).reshape(NW * NCH, CHUNK)
    ut2, it2 = _repack(user_table.T, item_table.T)
    ps, ns, uv, pv, nv = _bprmf(users2, pos2, neg2, ut2, it2)
    u_odd = (users.astype(jnp.int32) & 1)[:, None] == 1
    p_odd = (pos_items.astype(jnp.int32) & 1)[:, None] == 1
    n_odd = (neg_items.astype(jnp.int32) & 1)[:, None] == 1
    u_emb = jnp.where(u_odd, uv[:, D:], uv[:, :D])
    pos_emb = jnp.where(p_odd, pv[:, D:], pv[:, :D])
    neg_emb = jnp.where(n_odd, nv[:, D:], nv[:, :D])
    return (ps, ns, u_emb, pos_emb, neg_emb)


# speedup vs baseline: 3.8031x; 1.5398x over previous
import jax
import jax.numpy as jnp
from jax import lax
from jax.experimental import pallas as pl
from jax.experimental.pallas import tpu as pltpu
from jax.experimental.pallas import tpu_sc as plsc

B = 16384
D = 64
NC = 2
NS = 16
NW = NC * NS
BPW = B // NW
CHUNK = 128
NCH = BPW // CHUNK
GPC = CHUNK // 16

N_U = 1000000
N_I = 100000
CB_U = (N_U + 127) // 128
CB_I = (N_I + 127) // 128


def _transpose_stripe(colbuf, rowbuf):
    iota16 = lax.iota(jnp.int32, 16)
    lanes_l0 = [l0 + iota16 for l0 in range(0, 128, 16)]
    v_l0 = [(l0 + iota16) >> 1 for l0 in range(0, 128, 16)]
    hb = (iota16 & 1) * D

    def r_body(r, _):
        rowoff = (iota16 + r) & 15
        for d0 in range(0, D, 16):
            rows = rowoff + d0
            cc = hb + rows
            vals = [plsc.load_gather(colbuf, [rows, lanes_l0[q]])
                    for q in range(8)]
            for q in range(8):
                plsc.store_scatter(rowbuf, [v_l0[q], cc], vals[q])
        return 0

    lax.fori_loop(0, 16, r_body, 0)


def _repack_body(ut_t, it_t, u2_out, i2_out,
                 colA, colB, rowA, rowB, semA, semB):
    c = lax.axis_index("c")
    s = lax.axis_index("s")
    wid = s * NC + c

    def make_pair_loop(src, dst, n_full):
        def pair_body(tp, _):
            cb0 = wid + (2 * tp) * NW
            cb1 = wid + (2 * tp + 1) * NW
            cb0c = jnp.minimum(cb0, n_full - 1)
            cb1c = jnp.minimum(cb1, n_full - 1)
            st0 = pl.multiple_of(cb0c * 128, 128)
            st1 = pl.multiple_of(cb1c * 128, 128)
            ci0 = pltpu.async_copy(src.at[:, pl.ds(st0, 128)], colA, semA)
            ci1 = pltpu.async_copy(src.at[:, pl.ds(st1, 128)], colB, semB)
            ci0.wait()
            _transpose_stripe(colA, rowA)
            co0 = pltpu.async_copy(rowA, dst.at[pl.ds(cb0c * 64, 64)], semA)
            ci1.wait()
            _transpose_stripe(colB, rowB)
            co1 = pltpu.async_copy(rowB, dst.at[pl.ds(cb1c * 64, 64)], semB)
            co0.wait()
            co1.wait()
            return 0

        return pair_body

    n_full_u = N_U // 128
    n_full_i = N_I // 128
    lax.fori_loop(0, (n_full_u + 2 * NW - 1) // (2 * NW),
                  make_pair_loop(ut_t, u2_out, n_full_u), 0)
    lax.fori_loop(0, (n_full_i + 2 * NW - 1) // (2 * NW),
                  make_pair_loop(it_t, i2_out, n_full_i), 0)

    def tail(src, dst, cb, n_rows, who):
        @pl.when(wid == who)
        def _():
            start = pl.multiple_of(cb * 128, 128)
            pltpu.sync_copy(src.at[:, pl.ds(start, 128)], colA)
            _transpose_stripe(colA, rowA)
            nv = n_rows // 2 - cb * 64
            for q in range(4):
                if True:
                    @pl.when(nv >= (q + 1) * 16)
                    def _(q=q):
                        pltpu.sync_copy(rowA.at[pl.ds(q * 16, 16)],
                                        dst.at[pl.ds(cb * 64 + q * 16, 16)])

    tail(ut_t, u2_out, n_full_u, N_U, 0)
    tail(it_t, i2_out, n_full_i, N_I, 1)


@jax.jit
def _repack(ut_t, it_t):
    mesh = plsc.VectorSubcoreMesh(core_axis_name="c", subcore_axis_name="s",
                                  num_cores=NC, num_subcores=NS)
    f32 = jnp.float32
    run = pl.kernel(
        _repack_body,
        out_type=(jax.ShapeDtypeStruct((N_U // 2, 128), f32),
                  jax.ShapeDtypeStruct((N_I // 2, 128), f32)),
        mesh=mesh,
        scratch_types=[
            pltpu.VMEM((D, 128), f32),
            pltpu.VMEM((D, 128), f32),
            pltpu.VMEM((D, 128), f32),
            pltpu.VMEM((D, 128), f32),
            pltpu.SemaphoreType.DMA,
            pltpu.SemaphoreType.DMA,
        ],
        compiler_params=pltpu.CompilerParams(
            needs_layout_passes=False,
            use_tc_tiling_on_sc=True,
            disable_bounds_checks=True),
    )
    return run(ut_t, it_t)


def _bprmf_body(users_r, pos_r, neg_r, ut_r, it_r,
                pos_s_out, neg_s_out, u_out, p_out, n_out,
                idx_u, idx_p, idx_n, vr_u, vr_p, vr_n,
                u_rows, p_rows, n_rows, sc_p, sc_n, sem):
    c = lax.axis_index("c")
    s = lax.axis_index("s")
    wid = s * NC + c
    base = wid * BPW

    pltpu.sync_copy(users_r.at[pl.ds(wid * NCH, NCH)], idx_u)
    pltpu.sync_copy(pos_r.at[pl.ds(wid * NCH, NCH)], idx_p)
    pltpu.sync_copy(neg_r.at[pl.ds(wid * NCH, NCH)], idx_n)

    for ch in range(NCH):
        for g in range(GPC):
            sl = pl.ds(g * 16, 16)
            vr_u[ch, sl] = idx_u[ch, sl] >> 1
            vr_p[ch, sl] = idx_p[ch, sl] >> 1
            vr_n[ch, sl] = idx_n[ch, sl] >> 1

    iota16 = lax.iota(jnp.int32, 16)
    zero16 = jnp.zeros((16,), jnp.float32)

    for ch in range(NCH):
        cps = (pltpu.async_copy(ut_r.at[vr_u.at[ch]], u_rows, sem),
               pltpu.async_copy(it_r.at[vr_p.at[ch]], p_rows, sem),
               pltpu.async_copy(it_r.at[vr_n.at[ch]], n_rows, sem))
        for cp in cps:
            cp.wait()

        for g in range(GPC):
            sl = pl.ds(g * 16, 16)
            rows_e = g * 16 + iota16
            hu = (idx_u[ch, sl] & 1) * D
            hp = (idx_p[ch, sl] & 1) * D
            hn = (idx_n[ch, sl] & 1) * D

            def dbody(d, carry, rows_e=rows_e, hu=hu, hp=hp, hn=hn):
                ap, an = carry
                uc = plsc.load_gather(u_rows, [rows_e, hu + d])
                pc = plsc.load_gather(p_rows, [rows_e, hp + d])
                nc = plsc.load_gather(n_rows, [rows_e, hn + d])
                return (ap + uc * pc, an + uc * nc)

            ap, an = lax.fori_loop(0, D, dbody, (zero16, zero16))
            osl = pl.ds(ch * CHUNK + g * 16, 16)
            sc_p[osl] = ap
            sc_n[osl] = an

        out_sl = pl.ds(base + ch * CHUNK, CHUNK)
        pltpu.sync_copy(u_rows, u_out.at[out_sl])
        pltpu.sync_copy(p_rows, p_out.at[out_sl])
        pltpu.sync_copy(n_rows, n_out.at[out_sl])

    out_sl = pl.ds(base, BPW)
    pltpu.sync_copy(sc_p, pos_s_out.at[out_sl])
    pltpu.sync_copy(sc_n, neg_s_out.at[out_sl])


@jax.jit
def _bprmf(users2, pos2, neg2, ut2, it2):
    mesh = plsc.VectorSubcoreMesh(core_axis_name="c", subcore_axis_name="s",
                                  num_cores=NC, num_subcores=NS)
    f32 = jnp.float32
    out_type = (
        jax.ShapeDtypeStruct((B,), f32),
        jax.ShapeDtypeStruct((B,), f32),
        jax.ShapeDtypeStruct((B, 2 * D), f32),
        jax.ShapeDtypeStruct((B, 2 * D), f32),
        jax.ShapeDtypeStruct((B, 2 * D), f32),
    )
    i32 = jnp.int32
    scratch = [
        pltpu.VMEM((NCH, CHUNK), i32),
        pltpu.VMEM((NCH, CHUNK), i32),
        pltpu.VMEM((NCH, CHUNK), i32),
        pltpu.VMEM((NCH, CHUNK), i32),
        pltpu.VMEM((NCH, CHUNK), i32),
        pltpu.VMEM((NCH, CHUNK), i32),
        pltpu.VMEM((CHUNK, 2 * D), f32),
        pltpu.VMEM((CHUNK, 2 * D), f32),
        pltpu.VMEM((CHUNK, 2 * D), f32),
        pltpu.VMEM((BPW,), f32),
        pltpu.VMEM((BPW,), f32),
        pltpu.SemaphoreType.DMA,
    ]
    run = pl.kernel(_bprmf_body, out_type=out_type, mesh=mesh,
                    scratch_types=scratch,
                    compiler_params=pltpu.CompilerParams(
                        needs_layout_passes=False,
                        use_tc_tiling_on_sc=True))
    return run(users2, pos2, neg2, ut2, it2)


def kernel(users, pos_items, neg_items, user_table, item_table):
    users2 = users.astype(jnp.int32).reshape(NW * NCH, CHUNK)
    pos2 = pos_items.astype(jnp.int32).reshape(NW * NCH, CHUNK)
    neg2 = neg_items.astype(jnp.int32).reshape(NW * NCH, CHUNK)
    ut2, it2 = _repack(user_table.T, item_table.T)
    ps, ns, uv, pv, nv = _bprmf(users2, pos2, neg2, ut2, it2)
    u_odd = (users.astype(jnp.int32) & 1)[:, None] == 1
    p_odd = (pos_items.astype(jnp.int32) & 1)[:, None] == 1
    n_odd = (neg_items.astype(jnp.int32) & 1)[:, None] == 1
    u_emb = jnp.where(u_odd, uv[:, D:], uv[:, :D])
    pos_emb = jnp.where(p_odd, pv[:, D:], pv[:, :D])
    neg_emb = jnp.where(n_odd, nv[:, D:], nv[:, :D])
    return (ps, ns, u_emb, pos_emb, neg_emb)


# speedup vs baseline: 3.8258x; 1.0060x over previous
import jax
import jax.numpy as jnp
from jax import lax
from jax.experimental import pallas as pl
from jax.experimental.pallas import tpu as pltpu
from jax.experimental.pallas import tpu_sc as plsc

B = 16384
D = 64
NC = 2
NS = 16
NW = NC * NS
BPW = B // NW
CHUNK = 128
NCH = BPW // CHUNK
GPC = CHUNK // 16

N_U = 1000000
N_I = 100000
CB_U = (N_U + 127) // 128
CB_I = (N_I + 127) // 128


def _transpose_stripe(colbuf, rowbuf):
    iota16 = lax.iota(jnp.int32, 16)
    lanes_l0 = [l0 + iota16 for l0 in range(0, 128, 16)]
    v_l0 = [(l0 + iota16) >> 1 for l0 in range(0, 128, 16)]
    hb = (iota16 & 1) * D

    def r_body(r, _):
        for j in range(2):
            rowoff = (iota16 + (2 * r + j)) & 15
            for d0 in range(0, D, 16):
                rows = rowoff + d0
                cc = hb + rows
                vals = [plsc.load_gather(colbuf, [rows, lanes_l0[q]])
                        for q in range(8)]
                for q in range(8):
                    plsc.store_scatter(rowbuf, [v_l0[q], cc], vals[q])
        return 0

    lax.fori_loop(0, 8, r_body, 0)


def _repack_body(ut_t, it_t, u2_out, i2_out,
                 colA, colB, rowA, rowB, semA, semB):
    c = lax.axis_index("c")
    s = lax.axis_index("s")
    wid = s * NC + c

    def make_pair_loop(src, dst, n_full):
        def pair_body(tp, _):
            cb0 = wid + (2 * tp) * NW
            cb1 = wid + (2 * tp + 1) * NW
            cb0c = jnp.minimum(cb0, n_full - 1)
            cb1c = jnp.minimum(cb1, n_full - 1)
            st0 = pl.multiple_of(cb0c * 128, 128)
            st1 = pl.multiple_of(cb1c * 128, 128)
            ci0 = pltpu.async_copy(src.at[:, pl.ds(st0, 128)], colA, semA)
            ci1 = pltpu.async_copy(src.at[:, pl.ds(st1, 128)], colB, semB)
            ci0.wait()
            _transpose_stripe(colA, rowA)
            co0 = pltpu.async_copy(rowA, dst.at[pl.ds(cb0c * 64, 64)], semA)
            ci1.wait()
            _transpose_stripe(colB, rowB)
            co1 = pltpu.async_copy(rowB, dst.at[pl.ds(cb1c * 64, 64)], semB)
            co0.wait()
            co1.wait()
            return 0

        return pair_body

    n_full_u = N_U // 128
    n_full_i = N_I // 128
    lax.fori_loop(0, (n_full_u + 2 * NW - 1) // (2 * NW),
                  make_pair_loop(ut_t, u2_out, n_full_u), 0)
    lax.fori_loop(0, (n_full_i + 2 * NW - 1) // (2 * NW),
                  make_pair_loop(it_t, i2_out, n_full_i), 0)

    def tail(src, dst, cb, n_rows, who):
        @pl.when(wid == who)
        def _():
            start = pl.multiple_of(cb * 128, 128)
            pltpu.sync_copy(src.at[:, pl.ds(start, 128)], colA)
            _transpose_stripe(colA, rowA)
            nv = n_rows // 2 - cb * 64
            for q in range(4):
                if True:
                    @pl.when(nv >= (q + 1) * 16)
                    def _(q=q):
                        pltpu.sync_copy(rowA.at[pl.ds(q * 16, 16)],
                                        dst.at[pl.ds(cb * 64 + q * 16, 16)])

    tail(ut_t, u2_out, n_full_u, N_U, 0)
    tail(it_t, i2_out, n_full_i, N_I, 1)


@jax.jit
def _repack(ut_t, it_t):
    mesh = plsc.VectorSubcoreMesh(core_axis_name="c", subcore_axis_name="s",
                                  num_cores=NC, num_subcores=NS)
    f32 = jnp.float32
    run = pl.kernel(
        _repack_body,
        out_type=(jax.ShapeDtypeStruct((N_U // 2, 128), f32),
                  jax.ShapeDtypeStruct((N_I // 2, 128), f32)),
        mesh=mesh,
        scratch_types=[
            pltpu.VMEM((D, 128), f32),
            pltpu.VMEM((D, 128), f32),
            pltpu.VMEM((D, 128), f32),
            pltpu.VMEM((D, 128), f32),
            pltpu.SemaphoreType.DMA,
            pltpu.SemaphoreType.DMA,
        ],
        compiler_params=pltpu.CompilerParams(
            needs_layout_passes=False,
            use_tc_tiling_on_sc=True,
            disable_bounds_checks=True),
    )
    return run(ut_t, it_t)


def _bprmf_body(users_r, pos_r, neg_r, ut_r, it_r,
                pos_s_out, neg_s_out, u_out, p_out, n_out,
                idx_u, idx_p, idx_n, vr_u, vr_p, vr_n,
                u_rows, p_rows, n_rows, sc_p, sc_n, sem):
    c = lax.axis_index("c")
    s = lax.axis_index("s")
    wid = s * NC + c
    base = wid * BPW

    pltpu.sync_copy(users_r.at[pl.ds(wid * NCH, NCH)], idx_u)
    pltpu.sync_copy(pos_r.at[pl.ds(wid * NCH, NCH)], idx_p)
    pltpu.sync_copy(neg_r.at[pl.ds(wid * NCH, NCH)], idx_n)

    for ch in range(NCH):
        for g in range(GPC):
            sl = pl.ds(g * 16, 16)
            vr_u[ch, sl] = idx_u[ch, sl] >> 1
            vr_p[ch, sl] = idx_p[ch, sl] >> 1
            vr_n[ch, sl] = idx_n[ch, sl] >> 1

    iota16 = lax.iota(jnp.int32, 16)
    zero16 = jnp.zeros((16,), jnp.float32)

    for ch in range(NCH):
        cps = (pltpu.async_copy(ut_r.at[vr_u.at[ch]], u_rows, sem),
               pltpu.async_copy(it_r.at[vr_p.at[ch]], p_rows, sem),
               pltpu.async_copy(it_r.at[vr_n.at[ch]], n_rows, sem))
        for cp in cps:
            cp.wait()

        for g in range(GPC):
            sl = pl.ds(g * 16, 16)
            rows_e = g * 16 + iota16
            hu = (idx_u[ch, sl] & 1) * D
            hp = (idx_p[ch, sl] & 1) * D
            hn = (idx_n[ch, sl] & 1) * D

            def dbody(d, carry, rows_e=rows_e, hu=hu, hp=hp, hn=hn):
                ap, an = carry
                uc = plsc.load_gather(u_rows, [rows_e, hu + d])
                pc = plsc.load_gather(p_rows, [rows_e, hp + d])
                nc = plsc.load_gather(n_rows, [rows_e, hn + d])
                return (ap + uc * pc, an + uc * nc)

            ap, an = lax.fori_loop(0, D, dbody, (zero16, zero16))
            osl = pl.ds(ch * CHUNK + g * 16, 16)
            sc_p[osl] = ap
            sc_n[osl] = an

        out_sl = pl.ds(base + ch * CHUNK, CHUNK)
        pltpu.sync_copy(u_rows, u_out.at[out_sl])
        pltpu.sync_copy(p_rows, p_out.at[out_sl])
        pltpu.sync_copy(n_rows, n_out.at[out_sl])

    out_sl = pl.ds(base, BPW)
    pltpu.sync_copy(sc_p, pos_s_out.at[out_sl])
    pltpu.sync_copy(sc_n, neg_s_out.at[out_sl])


@jax.jit
def _bprmf(users2, pos2, neg2, ut2, it2):
    mesh = plsc.VectorSubcoreMesh(core_axis_name="c", subcore_axis_name="s",
                                  num_cores=NC, num_subcores=NS)
    f32 = jnp.float32
    out_type = (
        jax.ShapeDtypeStruct((B,), f32),
        jax.ShapeDtypeStruct((B,), f32),
        jax.ShapeDtypeStruct((B, 2 * D), f32),
        jax.ShapeDtypeStruct((B, 2 * D), f32),
        jax.ShapeDtypeStruct((B, 2 * D), f32),
    )
    i32 = jnp.int32
    scratch = [
        pltpu.VMEM((NCH, CHUNK), i32),
        pltpu.VMEM((NCH, CHUNK), i32),
        pltpu.VMEM((NCH, CHUNK), i32),
        pltpu.VMEM((NCH, CHUNK), i32),
        pltpu.VMEM((NCH, CHUNK), i32),
        pltpu.VMEM((NCH, CHUNK), i32),
        pltpu.VMEM((CHUNK, 2 * D), f32),
        pltpu.VMEM((CHUNK, 2 * D), f32),
        pltpu.VMEM((CHUNK, 2 * D), f32),
        pltpu.VMEM((BPW,), f32),
        pltpu.VMEM((BPW,), f32),
        pltpu.SemaphoreType.DMA,
    ]
    run = pl.kernel(_bprmf_body, out_type=out_type, mesh=mesh,
                    scratch_types=scratch,
                    compiler_params=pltpu.CompilerParams(
                        needs_layout_passes=False,
                        use_tc_tiling_on_sc=True))
    return run(users2, pos2, neg2, ut2, it2)


def kernel(users, pos_items, neg_items, user_table, item_table):
    users2 = users.astype(jnp.int32).reshape(NW * NCH, CHUNK)
    pos2 = pos_items.astype(jnp.int32).reshape(NW * NCH, CHUNK)
    neg2 = neg_items.astype(jnp.int32).reshape(NW * NCH, CHUNK)
    ut2, it2 = _repack(user_table.T, item_table.T)
    ps, ns, uv, pv, nv = _bprmf(users2, pos2, neg2, ut2, it2)
    u_odd = (users.astype(jnp.int32) & 1)[:, None] == 1
    p_odd = (pos_items.astype(jnp.int32) & 1)[:, None] == 1
    n_odd = (neg_items.astype(jnp.int32) & 1)[:, None] == 1
    u_emb = jnp.where(u_odd, uv[:, D:], uv[:, :D])
    pos_emb = jnp.where(p_odd, pv[:, D:], pv[:, :D])
    neg_emb = jnp.where(n_odd, nv[:, D:], nv[:, :D])
    return (ps, ns, u_emb, pos_emb, neg_emb)
